# baseline (speedup 1.0000x reference)
import functools

import jax
import jax.numpy as jnp
from jax import lax
from jax.experimental import pallas as pl
from jax.experimental.pallas import tpu as pltpu
from jax.experimental.pallas import tpu_sc as plsc

NC = 2
NS = 16
NW = NC * NS

CH = 64


def _sc_broadcast(B, L, D):
    rows_per_w = L // NW
    n_chunks = rows_per_w // CH
    mesh = plsc.VectorSubcoreMesh(core_axis_name="c", subcore_axis_name="s")

    @functools.partial(
        pl.kernel,
        mesh=mesh,
        out_type=jax.ShapeDtypeStruct((B, L, D), jnp.float32),
        scratch_types=[
            pltpu.VMEM((2, CH, D), jnp.float32),
            pltpu.SemaphoreType.DMA,
            pltpu.SemaphoreType.DMA,
        ],
    )
    def k(emb_hbm, out_hbm, buf, sem_in, sem_out):
        wid = lax.axis_index("s") * NC + lax.axis_index("c")
        base = wid * rows_per_w

        fetch = [
            pltpu.make_async_copy(
                emb_hbm.at[pl.ds(base + i * CH, CH), :], buf.at[i % 2], sem_in
            )
            for i in range(n_chunks)
        ]
        stores = [
            [
                pltpu.make_async_copy(
                    buf.at[i % 2],
                    out_hbm.at[b, pl.ds(base + i * CH, CH), :],
                    sem_out,
                )
                for b in range(B)
            ]
            for i in range(n_chunks)
        ]

        fetch[0].start()
        for i in range(n_chunks):
            fetch[i].wait()
            if i + 1 < n_chunks:
                if i >= 1:
                    for c in stores[i - 1]:
                        c.wait()
                fetch[i + 1].start()
            for c in stores[i]:
                c.start()
        for i in (n_chunks - 2, n_chunks - 1):
            if i >= 0:
                for c in stores[i]:
                    c.wait()

    return k


def kernel(x, emb):
    B, L = x.shape[0], x.shape[1]
    D = emb.shape[1]
    return _sc_broadcast(B, L, D)(emb)
